# baseline (speedup 1.0000x reference)
import jax
import jax.numpy as jnp
from jax.experimental import pallas as pl
from jax.experimental.pallas import tpu as pltpu

_B, _N, _D = 64, 512, 512
_BB = 4


def _gcn_kernel(h_ref, adj_in_ref, adj_out_ref, eb_ref, oeb_ref, gw_ref,
                gbias_ref, out_ref):
    eb = eb_ref[...]
    oeb = oeb_ref[...]
    adj_in = adj_in_ref[...]
    adj_out = adj_out_ref[...]
    gw = gw_ref[...]
    gbias = gbias_ref[...]
    for i in range(_BB):
        h = h_ref[i]
        g = jax.lax.dot_general(h, gw, (((1,), (1,)), ((), ())),
                                preferred_element_type=jnp.float32)
        s = jax.nn.sigmoid(g + gbias)
        s1 = s[:, 0:1]
        s2 = s[:, 1:2]
        s3 = s[:, 2:3]
        a = (s1 * adj_in + s2 * adj_out).astype(jnp.bfloat16)
        m = jnp.dot(a, h.astype(jnp.bfloat16),
                    preferred_element_type=jnp.float32)
        out_ref[i] = jnp.maximum(m + s1 * eb + s2 * oeb + s3 * h, 0.0)


def kernel(inputs, adj_in, edge_bias, gate_weight, bias_gate, adj_out,
           out_edge_bias, out_gate_weight, out_bias_gate, loop_gate):
    gw = jnp.concatenate(
        [gate_weight.T, out_gate_weight.T, loop_gate.T], axis=0)
    gbias = jnp.concatenate(
        [bias_gate, out_bias_gate, jnp.zeros_like(bias_gate)], axis=1)

    grid = (_B // _BB,)
    out = pl.pallas_call(
        _gcn_kernel,
        grid=grid,
        in_specs=[
            pl.BlockSpec((_BB, _N, _D), lambda b: (b, 0, 0)),
            pl.BlockSpec((_N, _N), lambda b: (0, 0)),
            pl.BlockSpec((_N, _N), lambda b: (0, 0)),
            pl.BlockSpec((_N, _D), lambda b: (0, 0)),
            pl.BlockSpec((_N, _D), lambda b: (0, 0)),
            pl.BlockSpec((3, _D), lambda b: (0, 0)),
            pl.BlockSpec((_N, 3), lambda b: (0, 0)),
        ],
        out_specs=pl.BlockSpec((_BB, _N, _D), lambda b: (b, 0, 0)),
        out_shape=jax.ShapeDtypeStruct((_B, _N, _D), jnp.float32),
        compiler_params=pltpu.CompilerParams(
            dimension_semantics=("parallel",)),
    )(inputs, adj_in, adj_out, edge_bias, out_edge_bias, gw, gbias)
    return out
